# baseline (speedup 1.0000x reference)
import functools

import jax
import jax.numpy as jnp
from jax import lax
from jax.experimental import pallas as pl
from jax.experimental.pallas import tpu as pltpu
from jax.experimental.pallas import tpu_sc as plsc

B = 64
K = 4
V = 100000
R = B * K
PLACEHOLDER = -1

BB = 16
RB = BB * K
VB = 25088
NC = (V + VB - 1) // VB

NW = 32
RPW = R // NW
LANES = 16
NV16 = V // LANES


def _sc_body(t2d_hbm, tflat_hbm, dflat_hbm, gidx_hbm, flags_hbm,
             pd_hbm, pt_hbm, tmax_hbm, tidx_hbm,
             idx_v, pd_v, pt_v, row_v, tmax_v, tidx_v, flags_v, sem):
    wid = lax.axis_index("s") * 2 + lax.axis_index("c")
    base = wid * RPW
    pltpu.sync_copy(flags_hbm.at[pl.ds(base, RPW)], flags_v.at[pl.ds(0, RPW)])
    fvec = flags_v[...]

    pltpu.sync_copy(gidx_hbm.at[pl.ds(base, RPW)], idx_v)
    pltpu.async_copy(tflat_hbm.at[idx_v], pt_v, sem).wait()
    pltpu.async_copy(dflat_hbm.at[idx_v], pd_v, sem).wait()
    pltpu.sync_copy(pt_v, pt_hbm.at[pl.ds(base, RPW)])
    pltpu.sync_copy(pd_v, pd_hbm.at[pl.ds(base, RPW)])

    lane_iota = lax.iota(jnp.int32, LANES)
    for r_local in range(RPW):
        row = base + r_local

        @pl.when(fvec[r_local] == 1)
        def _scan_row():
            pltpu.sync_copy(t2d_hbm.at[row], row_v)

            def step(j, carry):
                run_max, run_idx = carry
                v = lax.bitcast_convert_type(row_v[pl.ds(j * LANES, LANES)],
                                             jnp.int32)
                gvec = j * LANES + lane_iota
                better = v > run_max
                return (jnp.where(better, v, run_max),
                        jnp.where(better, gvec, run_idx))

            init = (jnp.full((LANES,), -1, jnp.int32),
                    jnp.zeros((LANES,), jnp.int32))
            run_max, run_idx = lax.fori_loop(0, NV16, step, init)
            tmax_v[r_local] = run_max
            tidx_v[r_local] = run_idx
    pltpu.sync_copy(tmax_v, tmax_hbm.at[pl.ds(base, RPW)])
    pltpu.sync_copy(tidx_v, tidx_hbm.at[pl.ds(base, RPW)])


_sc_call = functools.partial(
    pl.kernel,
    out_type=[
        jax.ShapeDtypeStruct((R,), jnp.float32),
        jax.ShapeDtypeStruct((R,), jnp.float32),
        jax.ShapeDtypeStruct((R, LANES), jnp.int32),
        jax.ShapeDtypeStruct((R, LANES), jnp.int32),
    ],
    mesh=plsc.VectorSubcoreMesh(core_axis_name="c", subcore_axis_name="s"),
    scratch_types=[
        pltpu.VMEM((RPW,), jnp.int32),
        pltpu.VMEM((RPW,), jnp.float32),
        pltpu.VMEM((RPW,), jnp.float32),
        pltpu.VMEM((V,), jnp.float32),
        pltpu.VMEM((RPW, LANES), jnp.int32),
        pltpu.VMEM((RPW, LANES), jnp.int32),
        pltpu.VMEM((LANES,), jnp.int32),
        pltpu.SemaphoreType.DMA,
    ],
)


def _scan_kernel(allg_ref, greedy_ref, t_ref, d_ref, q_ref, rec_ref,
                 run_rv, run_ri):
    del allg_ref
    c = pl.program_id(1)

    @pl.when(c == 0)
    def _init():
        run_rv[...] = jnp.full((RB, 1), -1, jnp.int32)
        run_ri[...] = jnp.zeros((RB, 1), jnp.int32)

    has_random = jnp.any(greedy_ref[...] == 0)

    @pl.when(has_random)
    def _random_path():
        t = t_ref[...]
        d = d_ref[...]
        qv = jnp.repeat(q_ref[...], K, axis=0)
        gid = jax.lax.broadcasted_iota(jnp.int32, (1, VB), 1) + c * VB
        ratio = jnp.maximum(t - d, 0.0) / qv
        rbits = jax.lax.bitcast_convert_type(ratio, jnp.int32)
        rbits = jnp.where(rbits < 0, jnp.int32(0x7FFFFFFF), rbits)
        rkey = jnp.where(gid < V, rbits, -1)
        bval = jnp.max(rkey, axis=1, keepdims=True)
        bidx = jnp.min(jnp.where(rkey == bval, gid, V), axis=1, keepdims=True)
        better = bval > run_rv[...]
        run_ri[...] = jnp.where(better, bidx, run_ri[...])
        run_rv[...] = jnp.maximum(run_rv[...], bval)

    @pl.when(c == NC - 1)
    def _emit():
        rec_ref[...] = run_ri[...]


def _combine_kernel(tok_ref, u_ref, bonus_ref, greedy_ref,
                    rec_ref, tmax_ref, tidx_ref, pd_ref, pt_ref, out_ref):
    pd = pd_ref[...]
    pt = pt_ref[...]
    u = u_ref[...]
    tok = tok_ref[...]
    bonus = bonus_ref[...]
    greedy = greedy_ref[...] != 0

    tmax = tmax_ref[...]
    tidx = tidx_ref[...]
    cols = []
    for k in range(K):
        mx = tmax[:, k * LANES:(k + 1) * LANES]
        ix = tidx[:, k * LANES:(k + 1) * LANES]
        m = jnp.max(mx, axis=1, keepdims=True)
        cols.append(jnp.min(jnp.where(mx == m, ix, V), axis=1, keepdims=True))
    t_arg = jnp.concatenate(cols, axis=1)

    r = jnp.where(pd > 0, pt / jnp.where(pd > 0, pd, 1.0), 0.0)
    accept = ((pd > 0) & (r >= u)).astype(jnp.int32)
    c1 = accept[:, 0:1]
    c2 = c1 * accept[:, 1:2]
    c3 = c2 * accept[:, 2:3]
    c4 = c3 * accept[:, 3:4]
    num_acc = c1 + c2 + c3 + c4

    pos = jax.lax.broadcasted_iota(jnp.int32, (B, K), 1)
    rand_tok = jnp.where(pos < num_acc, tok,
                         jnp.where(pos == num_acc, rec_ref[...], PLACEHOLDER))
    rand_bonus = jnp.where(num_acc == K, bonus, PLACEHOLDER)

    match = (tok == t_arg).astype(jnp.int32)
    m1 = match[:, 0:1]
    m2 = m1 * match[:, 1:2]
    m3 = m2 * match[:, 2:3]
    m4 = m3 * match[:, 3:4]
    num_match = m1 + m2 + m3 + m4
    greedy_tok = jnp.where(pos <= num_match, t_arg, PLACEHOLDER)
    greedy_bonus = jnp.where(num_match == K, bonus, PLACEHOLDER)

    out_tok = jnp.where(greedy, greedy_tok, rand_tok)
    out_bonus = jnp.where(greedy, greedy_bonus, rand_bonus)
    out_ref[...] = jnp.concatenate([out_tok, out_bonus], axis=1)


def kernel(draft_token_ids, cu_num_draft_tokens, draft_probs, target_probs,
           bonus_token_ids, uniform_probs, q, is_greedy):
    del cu_num_draft_tokens
    greedy_col = is_greedy.astype(jnp.int32).reshape(B, 1)
    all_greedy = jnp.all(is_greedy.reshape(B // BB, BB), axis=1).astype(jnp.int32)

    gidx = jnp.arange(R, dtype=jnp.int32) * V + draft_token_ids
    gflags = jnp.repeat(is_greedy.astype(jnp.int32), K)
    pd, pt, tmax, tidx = _sc_call(_sc_body)(
        target_probs, target_probs.reshape(R * V),
        draft_probs.reshape(R * V), gidx, gflags)

    def _skip_map(i, c, s):
        skip = s[i] == 1
        return jnp.where(skip, 0, i), jnp.where(skip, 0, c)

    rec = pl.pallas_call(
        _scan_kernel,
        grid_spec=pltpu.PrefetchScalarGridSpec(
            num_scalar_prefetch=1,
            grid=(B // BB, NC),
            in_specs=[
                pl.BlockSpec((BB, 1), lambda i, c, s: (i, 0)),
                pl.BlockSpec((RB, VB), _skip_map),
                pl.BlockSpec((RB, VB), _skip_map),
                pl.BlockSpec((BB, VB), _skip_map),
            ],
            out_specs=pl.BlockSpec((RB, 1), lambda i, c, s: (i, 0)),
            scratch_shapes=[
                pltpu.VMEM((RB, 1), jnp.int32),
                pltpu.VMEM((RB, 1), jnp.int32),
            ],
        ),
        out_shape=jax.ShapeDtypeStruct((R, 1), jnp.int32),
    )(all_greedy, greedy_col, target_probs, draft_probs, q)

    out = pl.pallas_call(
        _combine_kernel,
        out_shape=jax.ShapeDtypeStruct((B, K + 1), jnp.int32),
    )(draft_token_ids.reshape(B, K), uniform_probs.reshape(B, K),
      bonus_token_ids.reshape(B, 1), greedy_col,
      rec.reshape(B, K), tmax.reshape(B, K * LANES), tidx.reshape(B, K * LANES),
      pd.reshape(B, K), pt.reshape(B, K))
    return out


# speedup vs baseline: 1.2881x; 1.2881x over previous
import functools

import jax
import jax.numpy as jnp
from jax import lax
from jax.experimental import pallas as pl
from jax.experimental.pallas import tpu as pltpu
from jax.experimental.pallas import tpu_sc as plsc

B = 64
K = 4
V = 100000
R = B * K
PLACEHOLDER = -1

BB = 16
RB = BB * K
VB = 25088
NC = (V + VB - 1) // VB

NW = 32
RPW = R // NW
LANES = 16
NV16 = V // LANES


def _sc_body(t2d_hbm, tflat_hbm, dflat_hbm, gidx_hbm, flags_hbm,
             pd_hbm, pt_hbm, tmax_hbm, tidx_hbm,
             idx_v, pd_v, pt_v, row_v, tmax_v, tidx_v, flags_v, sem):
    wid = lax.axis_index("s") * 2 + lax.axis_index("c")
    base = wid * RPW
    pltpu.sync_copy(flags_hbm.at[pl.ds(base, RPW)], flags_v.at[pl.ds(0, RPW)])
    fvec = flags_v[...]

    pltpu.sync_copy(gidx_hbm.at[pl.ds(base, RPW)], idx_v)
    pltpu.async_copy(tflat_hbm.at[idx_v], pt_v, sem).wait()
    pltpu.async_copy(dflat_hbm.at[idx_v], pd_v, sem).wait()
    pltpu.sync_copy(pt_v, pt_hbm.at[pl.ds(base, RPW)])
    pltpu.sync_copy(pd_v, pd_hbm.at[pl.ds(base, RPW)])

    lane_iota = lax.iota(jnp.int32, LANES)
    QC = V // 4
    for r_local in range(RPW):
        row = wid + NW * r_local

        @pl.when(fvec[r_local] == 1)
        def _scan_row():
            cps = [pltpu.async_copy(
                       tflat_hbm.at[pl.ds(row * V + j * QC, QC)],
                       row_v.at[pl.ds(j * QC, QC)], sem)
                   for j in range(4)]
            for cp in cps:
                cp.wait()

            def step(j, carry):
                run_max, run_idx = carry
                for u in range(4):
                    off = j * 4 * LANES + u * LANES
                    v = lax.bitcast_convert_type(row_v[pl.ds(off, LANES)],
                                                 jnp.int32)
                    gvec = off + lane_iota
                    better = v > run_max
                    run_max = jnp.where(better, v, run_max)
                    run_idx = jnp.where(better, gvec, run_idx)
                return run_max, run_idx

            init = (jnp.full((LANES,), -1, jnp.int32),
                    jnp.zeros((LANES,), jnp.int32))
            run_max, run_idx = lax.fori_loop(0, NV16 // 4, step, init)
            tmax_v[r_local] = run_max
            tidx_v[r_local] = run_idx
    pltpu.sync_copy(tmax_v, tmax_hbm.at[pl.ds(base, RPW)])
    pltpu.sync_copy(tidx_v, tidx_hbm.at[pl.ds(base, RPW)])


_sc_call = functools.partial(
    pl.kernel,
    out_type=[
        jax.ShapeDtypeStruct((R,), jnp.float32),
        jax.ShapeDtypeStruct((R,), jnp.float32),
        jax.ShapeDtypeStruct((R, LANES), jnp.int32),
        jax.ShapeDtypeStruct((R, LANES), jnp.int32),
    ],
    mesh=plsc.VectorSubcoreMesh(core_axis_name="c", subcore_axis_name="s"),
    scratch_types=[
        pltpu.VMEM((RPW,), jnp.int32),
        pltpu.VMEM((RPW,), jnp.float32),
        pltpu.VMEM((RPW,), jnp.float32),
        pltpu.VMEM((V,), jnp.float32),
        pltpu.VMEM((RPW, LANES), jnp.int32),
        pltpu.VMEM((RPW, LANES), jnp.int32),
        pltpu.VMEM((LANES,), jnp.int32),
        pltpu.SemaphoreType.DMA,
    ],
)


def _scan_kernel(allg_ref, greedy_ref, t_ref, d_ref, q_ref, rec_ref,
                 run_rv, run_ri):
    del allg_ref
    c = pl.program_id(1)

    @pl.when(c == 0)
    def _init():
        run_rv[...] = jnp.full((RB, 1), -1, jnp.int32)
        run_ri[...] = jnp.zeros((RB, 1), jnp.int32)

    has_random = jnp.any(greedy_ref[...] == 0)

    @pl.when(has_random)
    def _random_path():
        t = t_ref[...]
        d = d_ref[...]
        qv = jnp.repeat(q_ref[...], K, axis=0)
        gid = jax.lax.broadcasted_iota(jnp.int32, (1, VB), 1) + c * VB
        ratio = jnp.maximum(t - d, 0.0) / qv
        rbits = jax.lax.bitcast_convert_type(ratio, jnp.int32)
        rbits = jnp.where(rbits < 0, jnp.int32(0x7FFFFFFF), rbits)
        rkey = jnp.where(gid < V, rbits, -1)
        bval = jnp.max(rkey, axis=1, keepdims=True)
        bidx = jnp.min(jnp.where(rkey == bval, gid, V), axis=1, keepdims=True)
        better = bval > run_rv[...]
        run_ri[...] = jnp.where(better, bidx, run_ri[...])
        run_rv[...] = jnp.maximum(run_rv[...], bval)

    @pl.when(c == NC - 1)
    def _emit():
        rec_ref[...] = run_ri[...]


def _combine_kernel(tok_ref, u_ref, bonus_ref, greedy_ref,
                    rec_ref, tmax_ref, tidx_ref, pd_ref, pt_ref, out_ref):
    pd = pd_ref[...]
    pt = pt_ref[...]
    u = u_ref[...]
    tok = tok_ref[...]
    bonus = bonus_ref[...]
    greedy = greedy_ref[...] != 0

    tmax = tmax_ref[...]
    tidx = tidx_ref[...]
    cols = []
    for k in range(K):
        mx = tmax[:, k * LANES:(k + 1) * LANES]
        ix = tidx[:, k * LANES:(k + 1) * LANES]
        m = jnp.max(mx, axis=1, keepdims=True)
        cols.append(jnp.min(jnp.where(mx == m, ix, V), axis=1, keepdims=True))
    t_arg = jnp.concatenate(cols, axis=1)

    r = jnp.where(pd > 0, pt / jnp.where(pd > 0, pd, 1.0), 0.0)
    accept = ((pd > 0) & (r >= u)).astype(jnp.int32)
    c1 = accept[:, 0:1]
    c2 = c1 * accept[:, 1:2]
    c3 = c2 * accept[:, 2:3]
    c4 = c3 * accept[:, 3:4]
    num_acc = c1 + c2 + c3 + c4

    pos = jax.lax.broadcasted_iota(jnp.int32, (B, K), 1)
    rand_tok = jnp.where(pos < num_acc, tok,
                         jnp.where(pos == num_acc, rec_ref[...], PLACEHOLDER))
    rand_bonus = jnp.where(num_acc == K, bonus, PLACEHOLDER)

    match = (tok == t_arg).astype(jnp.int32)
    m1 = match[:, 0:1]
    m2 = m1 * match[:, 1:2]
    m3 = m2 * match[:, 2:3]
    m4 = m3 * match[:, 3:4]
    num_match = m1 + m2 + m3 + m4
    greedy_tok = jnp.where(pos <= num_match, t_arg, PLACEHOLDER)
    greedy_bonus = jnp.where(num_match == K, bonus, PLACEHOLDER)

    out_tok = jnp.where(greedy, greedy_tok, rand_tok)
    out_bonus = jnp.where(greedy, greedy_bonus, rand_bonus)
    out_ref[...] = jnp.concatenate([out_tok, out_bonus], axis=1)


def kernel(draft_token_ids, cu_num_draft_tokens, draft_probs, target_probs,
           bonus_token_ids, uniform_probs, q, is_greedy):
    del cu_num_draft_tokens
    greedy_col = is_greedy.astype(jnp.int32).reshape(B, 1)
    all_greedy = jnp.all(is_greedy.reshape(B // BB, BB), axis=1).astype(jnp.int32)

    gidx = jnp.arange(R, dtype=jnp.int32) * V + draft_token_ids
    gflags = jnp.repeat(is_greedy.astype(jnp.int32), K)
    gflags = gflags.reshape(RPW, NW).T.reshape(R)
    pd, pt, tmax_s, tidx_s = _sc_call(_sc_body)(
        target_probs, target_probs.reshape(R * V),
        draft_probs.reshape(R * V), gidx, gflags)
    tmax = tmax_s.reshape(NW, RPW, LANES).transpose(1, 0, 2).reshape(R, LANES)
    tidx = tidx_s.reshape(NW, RPW, LANES).transpose(1, 0, 2).reshape(R, LANES)

    def _skip_map(i, c, s):
        skip = s[i] == 1
        return jnp.where(skip, 0, i), jnp.where(skip, 0, c)

    rec = pl.pallas_call(
        _scan_kernel,
        grid_spec=pltpu.PrefetchScalarGridSpec(
            num_scalar_prefetch=1,
            grid=(B // BB, NC),
            in_specs=[
                pl.BlockSpec((BB, 1), lambda i, c, s: (i, 0)),
                pl.BlockSpec((RB, VB), _skip_map),
                pl.BlockSpec((RB, VB), _skip_map),
                pl.BlockSpec((BB, VB), _skip_map),
            ],
            out_specs=pl.BlockSpec((RB, 1), lambda i, c, s: (i, 0)),
            scratch_shapes=[
                pltpu.VMEM((RB, 1), jnp.int32),
                pltpu.VMEM((RB, 1), jnp.int32),
            ],
        ),
        out_shape=jax.ShapeDtypeStruct((R, 1), jnp.int32),
    )(all_greedy, greedy_col, target_probs, draft_probs, q)

    out = pl.pallas_call(
        _combine_kernel,
        out_shape=jax.ShapeDtypeStruct((B, K + 1), jnp.int32),
    )(draft_token_ids.reshape(B, K), uniform_probs.reshape(B, K),
      bonus_token_ids.reshape(B, 1), greedy_col,
      rec.reshape(B, K), tmax.reshape(B, K * LANES), tidx.reshape(B, K * LANES),
      pd.reshape(B, K), pt.reshape(B, K))
    return out


# speedup vs baseline: 2.4771x; 1.9230x over previous
import jax
import jax.numpy as jnp
from jax.experimental import pallas as pl
from jax.experimental.pallas import tpu as pltpu

B = 64
K = 4
V = 100000
PLACEHOLDER = -1

BB = 16
RB = BB * K
VB = 25088
NC = (V + VB - 1) // VB


def _scan_kernel(allg_ref, greedy_ref, tok_ref, t_ref, d_ref, q_ref,
                 rec_ref, targ_ref, pd_ref, pt_ref,
                 run_rv, run_ri, run_tv, run_ti, acc_pd, acc_pt):
    del allg_ref
    c = pl.program_id(1)

    @pl.when(c == 0)
    def _init():
        run_rv[...] = jnp.full((RB, 1), -1, jnp.int32)
        run_ri[...] = jnp.zeros((RB, 1), jnp.int32)
        run_tv[...] = jnp.full((RB, 1), -1, jnp.int32)
        run_ti[...] = jnp.zeros((RB, 1), jnp.int32)
        acc_pd[...] = jnp.zeros((RB, 1), jnp.float32)
        acc_pt[...] = jnp.zeros((RB, 1), jnp.float32)

    g = greedy_ref[...] != 0
    has_greedy = jnp.any(g)
    has_random = jnp.any(jnp.logical_not(g))

    t = t_ref[...]
    gid = jax.lax.broadcasted_iota(jnp.int32, (1, VB), 1) + c * VB
    valid = gid < V

    def upd_argmax(x, rv_ref, ri_ref):
        bval = jnp.max(x, axis=1, keepdims=True)
        eq = x == bval
        bidx = jnp.min(jnp.where(eq, gid, V), axis=1, keepdims=True)
        better = bval > rv_ref[...]
        ri_ref[...] = jnp.where(better, bidx, ri_ref[...])
        rv_ref[...] = jnp.maximum(rv_ref[...], bval)

    @pl.when(has_random)
    def _random_path():
        d = d_ref[...]
        qv = jnp.repeat(q_ref[...], K, axis=0)
        ratio = jnp.maximum(t - d, 0.0) / qv
        rbits = jax.lax.bitcast_convert_type(ratio, jnp.int32)
        rbits = jnp.where(rbits < 0, jnp.int32(0x7FFFFFFF), rbits)
        rkey = jnp.where(valid, rbits, -1)
        upd_argmax(rkey, run_rv, run_ri)
        tok = tok_ref[...]
        hit = gid == tok
        acc_pt[...] += jnp.sum(jnp.where(hit, t, 0.0), axis=1, keepdims=True)
        acc_pd[...] += jnp.sum(jnp.where(hit, d, 0.0), axis=1, keepdims=True)

    @pl.when(has_greedy)
    def _greedy_path():
        tkey = jnp.where(valid, jax.lax.bitcast_convert_type(t, jnp.int32), -1)
        upd_argmax(tkey, run_tv, run_ti)

    @pl.when(c == NC - 1)
    def _emit():
        rec_ref[...] = run_ri[...]
        targ_ref[...] = run_ti[...]
        pd_ref[...] = acc_pd[...]
        pt_ref[...] = acc_pt[...]


def _combine_kernel(tok_ref, u_ref, bonus_ref, greedy_ref,
                    rec_ref, targ_ref, pd_ref, pt_ref, out_ref):
    pd = pd_ref[...]
    pt = pt_ref[...]
    u = u_ref[...]
    tok = tok_ref[...]
    bonus = bonus_ref[...]
    greedy = greedy_ref[...] != 0

    r = jnp.where(pd > 0, pt / jnp.where(pd > 0, pd, 1.0), 0.0)
    accept = ((pd > 0) & (r >= u)).astype(jnp.int32)
    c1 = accept[:, 0:1]
    c2 = c1 * accept[:, 1:2]
    c3 = c2 * accept[:, 2:3]
    c4 = c3 * accept[:, 3:4]
    num_acc = c1 + c2 + c3 + c4

    pos = jax.lax.broadcasted_iota(jnp.int32, (B, K), 1)
    rand_tok = jnp.where(pos < num_acc, tok,
                         jnp.where(pos == num_acc, rec_ref[...], PLACEHOLDER))
    rand_bonus = jnp.where(num_acc == K, bonus, PLACEHOLDER)

    t_arg = targ_ref[...]
    match = (tok == t_arg).astype(jnp.int32)
    m1 = match[:, 0:1]
    m2 = m1 * match[:, 1:2]
    m3 = m2 * match[:, 2:3]
    m4 = m3 * match[:, 3:4]
    num_match = m1 + m2 + m3 + m4
    greedy_tok = jnp.where(pos <= num_match, t_arg, PLACEHOLDER)
    greedy_bonus = jnp.where(num_match == K, bonus, PLACEHOLDER)

    out_tok = jnp.where(greedy, greedy_tok, rand_tok)
    out_bonus = jnp.where(greedy, greedy_bonus, rand_bonus)
    out_ref[...] = jnp.concatenate([out_tok, out_bonus], axis=1)


def kernel(draft_token_ids, cu_num_draft_tokens, draft_probs, target_probs,
           bonus_token_ids, uniform_probs, q, is_greedy):
    del cu_num_draft_tokens
    tok_col = draft_token_ids.reshape(B * K, 1)
    greedy_col = is_greedy.astype(jnp.int32).reshape(B, 1)
    all_greedy = jnp.all(is_greedy.reshape(B // BB, BB), axis=1).astype(jnp.int32)

    def _dq_map(i, c, s):
        skip = s[i] == 1
        return jnp.where(skip, 0, i), jnp.where(skip, 0, c)

    small = jax.ShapeDtypeStruct((B * K, 1), jnp.int32)
    smallf = jax.ShapeDtypeStruct((B * K, 1), jnp.float32)
    rec, targ, pd, pt = pl.pallas_call(
        _scan_kernel,
        grid_spec=pltpu.PrefetchScalarGridSpec(
            num_scalar_prefetch=1,
            grid=(B // BB, NC),
            in_specs=[
                pl.BlockSpec((BB, 1), lambda i, c, s: (i, 0)),
                pl.BlockSpec((RB, 1), lambda i, c, s: (i, 0)),
                pl.BlockSpec((RB, VB), lambda i, c, s: (i, c)),
                pl.BlockSpec((RB, VB), _dq_map),
                pl.BlockSpec((BB, VB), _dq_map),
            ],
            out_specs=[
                pl.BlockSpec((RB, 1), lambda i, c, s: (i, 0)),
                pl.BlockSpec((RB, 1), lambda i, c, s: (i, 0)),
                pl.BlockSpec((RB, 1), lambda i, c, s: (i, 0)),
                pl.BlockSpec((RB, 1), lambda i, c, s: (i, 0)),
            ],
            scratch_shapes=[
                pltpu.VMEM((RB, 1), jnp.int32),
                pltpu.VMEM((RB, 1), jnp.int32),
                pltpu.VMEM((RB, 1), jnp.int32),
                pltpu.VMEM((RB, 1), jnp.int32),
                pltpu.VMEM((RB, 1), jnp.float32),
                pltpu.VMEM((RB, 1), jnp.float32),
            ],
        ),
        out_shape=[small, small, smallf, smallf],
    )(all_greedy, greedy_col, tok_col, target_probs, draft_probs, q)

    out = pl.pallas_call(
        _combine_kernel,
        out_shape=jax.ShapeDtypeStruct((B, K + 1), jnp.int32),
    )(draft_token_ids.reshape(B, K), uniform_probs.reshape(B, K),
      bonus_token_ids.reshape(B, 1), greedy_col,
      rec.reshape(B, K), targ.reshape(B, K),
      pd.reshape(B, K), pt.reshape(B, K))
    return out
